# initial kernel scaffold (unmeasured)
import jax
import jax.numpy as jnp
from jax import lax
from jax.experimental import pallas as pl
from jax.experimental.pallas import tpu as pltpu


def kernel(
    x,
):
    def body(*refs):
        pass

    out_shape = jax.ShapeDtypeStruct(..., jnp.float32)
    return pl.pallas_call(body, out_shape=out_shape)(...)


# baseline (device time: 10069 ns/iter reference)
import jax
import jax.numpy as jnp
from jax import lax
from jax.experimental import pallas as pl
from jax.experimental.pallas import tpu as pltpu

N_DEV = 4


def kernel(x):
    m_per, n = x.shape
    m_total = N_DEV * m_per
    log_m = m_total.bit_length() - 1
    assert (1 << log_m) == m_total

    def body(x_ref, out_ref, gbuf, sorted_ref, send_sems, recv_sems):
        my = lax.axis_index("i")

        barrier_sem = pltpu.get_barrier_semaphore()
        for d in range(1, N_DEV):
            pl.semaphore_signal(
                barrier_sem,
                inc=1,
                device_id=((my + d) % N_DEV,),
                device_id_type=pl.DeviceIdType.MESH,
            )
        pl.semaphore_wait(barrier_sem, N_DEV - 1)

        gbuf[0] = x_ref[...].astype(jnp.bfloat16)

        rdmas = []
        for d in range(1, N_DEV):
            rdma = pltpu.make_async_remote_copy(
                src_ref=gbuf.at[0],
                dst_ref=gbuf.at[d],
                send_sem=send_sems.at[d - 1],
                recv_sem=recv_sems.at[d - 1],
                device_id=((my + d) % N_DEV,),
                device_id_type=pl.DeviceIdType.MESH,
            )
            rdma.start()
            rdmas.append(rdma)
        for rdma in rdmas:
            rdma.wait()

        v = jnp.concatenate([gbuf[d] for d in range(N_DEV)], axis=0)
        rows = lax.broadcasted_iota(jnp.int32, (m_total, n), 0)
        for s in range(log_m):
            kk = 1 << (s + 1)
            j = kk >> 1
            while j >= 1:
                down = jnp.concatenate([v[j:], v[:j]], axis=0)
                up = jnp.concatenate([v[m_total - j:], v[:m_total - j]], axis=0)
                w = jnp.where((rows & j) == 0, down, up)
                take_max = ((rows & j) != 0) != ((rows & kk) != 0)
                v = jnp.where(take_max, jnp.maximum(v, w), jnp.minimum(v, w))
                j >>= 1

        sorted_ref[...] = v.astype(jnp.float32)
        out_ref[...] = sorted_ref[pl.ds(my * m_per, m_per), :]

    return pl.pallas_call(
        body,
        out_shape=jax.ShapeDtypeStruct((m_per, n), jnp.float32),
        in_specs=[pl.BlockSpec(memory_space=pltpu.VMEM)],
        out_specs=pl.BlockSpec(memory_space=pltpu.VMEM),
        scratch_shapes=[
            pltpu.VMEM((N_DEV, m_per, n), jnp.bfloat16),
            pltpu.VMEM((m_total, n), jnp.float32),
            pltpu.SemaphoreType.DMA((N_DEV - 1,)),
            pltpu.SemaphoreType.DMA((N_DEV - 1,)),
        ],
        compiler_params=pltpu.CompilerParams(collective_id=0),
    )(x)


# device time: 8194 ns/iter; 1.2288x vs baseline; 1.2288x over previous
import jax
import jax.numpy as jnp
from jax import lax
from jax.experimental import pallas as pl
from jax.experimental.pallas import tpu as pltpu

N_DEV = 4


def _cmp_stage(v, rows, j, take_max):
    m = v.shape[0]
    down = jnp.concatenate([v[j:], v[:j]], axis=0)
    up = jnp.concatenate([v[m - j:], v[:m - j]], axis=0)
    w = jnp.where((rows & j) == 0, down, up)
    return jnp.where(take_max, jnp.maximum(v, w), jnp.minimum(v, w))


def kernel(x):
    m_per, n = x.shape
    m_total = N_DEV * m_per
    log_per = m_per.bit_length() - 1
    log_tot = m_total.bit_length() - 1
    assert (1 << log_per) == m_per and (1 << log_tot) == m_total

    def body(x_ref, out_ref, gbuf, sorted_ref, send_sems, recv_sems):
        my = lax.axis_index("i")
        my_odd = (my % 2) == 1

        barrier_sem = pltpu.get_barrier_semaphore()
        for d in range(1, N_DEV):
            pl.semaphore_signal(
                barrier_sem,
                inc=1,
                device_id=((my + d) % N_DEV,),
                device_id_type=pl.DeviceIdType.MESH,
            )

        v = x_ref[...].astype(jnp.bfloat16)
        rows_l = lax.broadcasted_iota(jnp.int32, (m_per, n), 0)
        for s in range(log_per):
            kk = 1 << (s + 1)
            j = kk >> 1
            while j >= 1:
                tm = ((rows_l & j) != 0) != ((rows_l & kk) != 0)
                if kk == m_per:
                    tm = tm != my_odd
                v = _cmp_stage(v, rows_l, j, tm)
                j >>= 1
        gbuf[my] = v

        pl.semaphore_wait(barrier_sem, N_DEV - 1)

        rdmas = []
        for d in range(1, N_DEV):
            rdma = pltpu.make_async_remote_copy(
                src_ref=gbuf.at[my],
                dst_ref=gbuf.at[my],
                send_sem=send_sems.at[d - 1],
                recv_sem=recv_sems.at[d - 1],
                device_id=((my + d) % N_DEV,),
                device_id_type=pl.DeviceIdType.MESH,
            )
            rdma.start()
            rdmas.append(rdma)
        for rdma in rdmas:
            rdma.wait()

        v = jnp.concatenate([gbuf[d] for d in range(N_DEV)], axis=0)
        rows = lax.broadcasted_iota(jnp.int32, (m_total, n), 0)
        for s in range(log_per, log_tot):
            kk = 1 << (s + 1)
            j = kk >> 1
            while j >= 1:
                tm = ((rows & j) != 0) != ((rows & kk) != 0)
                v = _cmp_stage(v, rows, j, tm)
                j >>= 1

        sorted_ref[...] = v.astype(jnp.float32)
        out_ref[...] = sorted_ref[pl.ds(my * m_per, m_per), :]

    return pl.pallas_call(
        body,
        out_shape=jax.ShapeDtypeStruct((m_per, n), jnp.float32),
        in_specs=[pl.BlockSpec(memory_space=pltpu.VMEM)],
        out_specs=pl.BlockSpec(memory_space=pltpu.VMEM),
        scratch_shapes=[
            pltpu.VMEM((N_DEV, m_per, n), jnp.bfloat16),
            pltpu.VMEM((m_total, n), jnp.float32),
            pltpu.SemaphoreType.DMA((N_DEV - 1,)),
            pltpu.SemaphoreType.DMA((N_DEV - 1,)),
        ],
        compiler_params=pltpu.CompilerParams(collective_id=0),
    )(x)
